# baseline (speedup 1.0000x reference)
import jax
import jax.numpy as jnp
from jax import lax
from jax.experimental import pallas as pl
from jax.experimental.pallas import tpu as pltpu
from jax.experimental.pallas import tpu_sc as plsc

NUM_CORES = 2
NUM_SUBCORES = 16
LANES = 16
NW = NUM_CORES * NUM_SUBCORES
BATCH = 16384
BPW = BATCH // NW
TPW = BPW // 128

GAMMA = 0.25
ALPHA = GAMMA * 4.0
BETA = ALPHA * 200.0 / 20.0


def _log1p(t):
    z = t / (t + 2.0)
    z2 = z * z
    p = 1.0 / 9.0
    for c in (1.0 / 7.0, 1.0 / 5.0, 1.0 / 3.0, 1.0):
        p = c + z2 * p
    return 2.0 * z * p


def _logsig(x):
    return jnp.minimum(x, 0.0) - _log1p(jnp.exp(-jnp.abs(x)))


def _body(inst, prd, ie, qi, qp, pred_out, q_out, reg_out,
          idx_v, pidx_v, idxe, idxq, qp_tab, po_v, qi_v,
          acc_v, red_v, out_v, shared, sem, sem2):
    cid = lax.axis_index("c")
    sid = lax.axis_index("s")
    wid = sid * NUM_CORES + cid
    base = wid * BPW

    cp_tab = pltpu.async_copy(qp, qp_tab, sem2)
    with jax.named_scope("idx_stage"):
        cp_i = pltpu.async_copy(inst.at[pl.ds(base, BPW)], idx_v, sem)
        cp_p = pltpu.async_copy(prd.at[pl.ds(base, BPW)], pidx_v, sem)
        cp_i.wait()
        cp_p.wait()

    def build(k, carry):
        t = k // 8
        u = k - t * 8
        pe = t * 256 + u * LANES
        pq = t * 512 + u * LANES
        v = idx_v[pl.ds(k * LANES, LANES)]
        p = pidx_v[pl.ds(k * LANES, LANES)]
        idxe[pl.ds(pe, LANES)] = v
        idxe[pl.ds(pe + 128, LANES)] = v + 100000
        for c1 in range(2):
            for c2 in range(2):
                sl = pl.ds(c1 * 1024 + t * 256 + c2 * 128 + u * LANES, LANES)
                idxq[sl] = v + (2 * c1 + c2) * 100000
        return carry

    with jax.named_scope("build"):
        lax.fori_loop(0, BPW // LANES, build, 0)

    with jax.named_scope("gather"):
        cp_e = pltpu.async_copy(ie.at[idxe], po_v, sem)
        cp_q = pltpu.async_copy(qi.at[idxq], qi_v, sem)
        cp_e.wait()

    def pred_step(k, carry):
        sl = pl.ds(k * LANES, LANES)
        po_v[sl] = _logsig(po_v[sl])
        return carry

    with jax.named_scope("pred"):
        lax.fori_loop(0, 2 * BPW // LANES, pred_step, 0)

    with jax.named_scope("gather_q"):
        cp_q.wait()
        cp_tab.wait()

    def compute(k, carry):
        acc_i, acc_p = carry
        t = k // 8
        u = k - t * 8
        pq = t * 256 + u * LANES

        q0 = pl.ds(pq, LANES)
        q1 = pl.ds(pq + 128, LANES)
        q2 = pl.ds(pq + 1024, LANES)
        q3 = pl.ds(pq + 1152, LANES)
        a0 = qi_v[q0]
        a1 = qi_v[q1]
        a2 = qi_v[q2]
        a3 = qi_v[q3]
        pld = pidx_v[pl.ds(k * LANES, LANES)]
        b0 = plsc.load_gather(qp_tab, [pld])
        b1 = plsc.load_gather(qp_tab, [pld + 1000])
        b2 = plsc.load_gather(qp_tab, [pld + 2000])
        b3 = plsc.load_gather(qp_tab, [pld + 3000])
        acc_i = acc_i + a0 * a0 + a1 * a1 + a2 * a2 + a3 * a3
        acc_p = acc_p + b0 * b0 + b1 * b1 + b2 * b2 + b3 * b3
        x0 = a0 + b0
        x1 = a1 + b1
        x2 = a2 + b2
        x3 = a3 + b3
        m01 = jnp.maximum(x0, x1)
        l01 = m01 + _log1p(jnp.exp(jnp.minimum(x0, x1) - m01))
        m23 = jnp.maximum(x2, x3)
        l23 = m23 + _log1p(jnp.exp(jnp.minimum(x2, x3) - m23))
        qi_v[q0] = x0 - l01
        qi_v[q1] = x1 - l01
        qi_v[q2] = x2 - l23
        qi_v[q3] = x3 - l23
        return acc_i, acc_p

    zero = jnp.zeros((LANES,), jnp.float32)
    with jax.named_scope("compute"):
        acc_i, acc_p = lax.fori_loop(0, BPW // LANES, compute, (zero, zero))

    pltpu.sync_copy(po_v, pred_out.at[pl.ds(wid * 2 * BPW, 2 * BPW)])
    pltpu.sync_copy(qi_v.at[pl.ds(0, 2 * BPW)], q_out.at[pl.ds(wid * 2 * BPW, 2 * BPW)])
    pltpu.sync_copy(qi_v.at[pl.ds(2 * BPW, 2 * BPW)],
                    q_out.at[pl.ds(2 * BATCH + wid * 2 * BPW, 2 * BPW)])

    acc_v[0, :] = acc_i * (BETA * 0.5)
    acc_v[1, :] = acc_p * (ALPHA * 0.5)
    pltpu.sync_copy(acc_v, shared.at[sid])
    plsc.subcore_barrier()

    @pl.when(sid == 0)
    def _():
        pltpu.sync_copy(shared, red_v)
        ti = red_v[0, 0, :]
        tp = red_v[0, 1, :]
        for s in range(1, NUM_SUBCORES):
            ti = ti + red_v[s, 0, :]
            tp = tp + red_v[s, 1, :]
        out_v[0, :] = jnp.full((LANES,), jnp.sum(ti), jnp.float32)
        out_v[1, :] = jnp.full((LANES,), jnp.sum(tp), jnp.float32)
        pltpu.sync_copy(out_v, reg_out.at[cid])


_sc_call = pl.kernel(
    _body,
    mesh=plsc.VectorSubcoreMesh(core_axis_name="c", subcore_axis_name="s"),
    compiler_params=pltpu.CompilerParams(
        needs_layout_passes=False,
        use_tc_tiling_on_sc=False,
        skip_device_barrier=True,
    ),
    out_type=[
        jax.ShapeDtypeStruct((BATCH * 2,), jnp.float32),
        jax.ShapeDtypeStruct((BATCH * 4,), jnp.float32),
        jax.ShapeDtypeStruct((NUM_CORES, 2, LANES), jnp.float32),
    ],
    scratch_types=[
        pltpu.VMEM((BPW,), jnp.int32),
        pltpu.VMEM((BPW,), jnp.int32),
        pltpu.VMEM((2 * BPW,), jnp.int32),
        pltpu.VMEM((4 * BPW,), jnp.int32),
        pltpu.VMEM((4000,), jnp.float32),
        pltpu.VMEM((2 * BPW,), jnp.float32),
        pltpu.VMEM((4 * BPW,), jnp.float32),
        pltpu.VMEM((2, LANES), jnp.float32),
        pltpu.VMEM((NUM_SUBCORES, 2, LANES), jnp.float32),
        pltpu.VMEM((2, LANES), jnp.float32),
        pltpu.VMEM_SHARED((NUM_SUBCORES, 2, LANES), jnp.float32),
        pltpu.SemaphoreType.DMA,
        pltpu.SemaphoreType.DMA,
    ],
)


def kernel(instances, predictors, labels, instance_emb, q_i_emb, q_p_emb):
    del labels
    pred, q, reg = _sc_call(
        instances, predictors,
        instance_emb.T.reshape(-1), q_i_emb.T.reshape(-1), q_p_emb.T.reshape(-1),
    )
    predictions = pred.reshape(128, 2, 128).transpose(0, 2, 1).reshape(BATCH, 2)
    q_params = q.reshape(2, 128, 2, 128).transpose(1, 3, 0, 2).reshape(BATCH, 2, 2)
    reg_i = reg[0, 0, 0] + reg[1, 0, 0]
    reg_p = reg[0, 1, 0] + reg[1, 1, 0]
    return predictions, q_params, reg_i, reg_p


# speedup vs baseline: 1.0198x; 1.0198x over previous
import jax
import jax.numpy as jnp
from jax import lax
from jax.experimental import pallas as pl
from jax.experimental.pallas import tpu as pltpu
from jax.experimental.pallas import tpu_sc as plsc

NUM_CORES = 2
NUM_SUBCORES = 16
LANES = 16
NW = NUM_CORES * NUM_SUBCORES
BATCH = 16384
BPW = BATCH // NW
TPW = BPW // 128

GAMMA = 0.25
ALPHA = GAMMA * 4.0
BETA = ALPHA * 200.0 / 20.0


def _log1p(t):
    z = t / (t + 2.0)
    z2 = z * z
    p = 1.0 / 9.0
    for c in (1.0 / 7.0, 1.0 / 5.0, 1.0 / 3.0, 1.0):
        p = c + z2 * p
    return 2.0 * z * p


def _logsig(x):
    return jnp.minimum(x, 0.0) - _log1p(jnp.exp(-jnp.abs(x)))


def _body(inst, prd, ie, qi, qp, pred_out, q_out, reg_out,
          idx_v, pidx_v, idxe, idxq, qp_tab, po_v, qi_v,
          acc_v, red_v, out_v, shared, sem, sem2):
    cid = lax.axis_index("c")
    sid = lax.axis_index("s")
    wid = sid * NUM_CORES + cid
    base = wid * BPW

    cp_tab = pltpu.async_copy(qp, qp_tab, sem2)
    with jax.named_scope("idx_stage"):
        cp_i = pltpu.async_copy(inst.at[pl.ds(base, BPW)], idx_v, sem)
        cp_p = pltpu.async_copy(prd.at[pl.ds(base, BPW)], pidx_v, sem)
        cp_i.wait()
        cp_p.wait()

    def build(k, carry):
        t = k // 8
        u = k - t * 8
        pe = t * 256 + u * LANES
        pq = t * 512 + u * LANES
        v = idx_v[pl.ds(k * LANES, LANES)]
        p = pidx_v[pl.ds(k * LANES, LANES)]
        idxe[pl.ds(pe, LANES)] = v
        idxe[pl.ds(pe + 128, LANES)] = v + 100000
        for c1 in range(2):
            for c2 in range(2):
                sl = pl.ds(c1 * 1024 + t * 256 + c2 * 128 + u * LANES, LANES)
                idxq[sl] = v + (2 * c1 + c2) * 100000
        return carry

    with jax.named_scope("build"):
        lax.fori_loop(0, BPW // LANES, build, 0, unroll=2)

    half = BPW
    with jax.named_scope("gather"):
        cp_e1 = pltpu.async_copy(ie.at[idxe.at[pl.ds(0, half)]],
                                 po_v.at[pl.ds(0, half)], sem)
        cp_e2 = pltpu.async_copy(ie.at[idxe.at[pl.ds(half, half)]],
                                 po_v.at[pl.ds(half, half)], sem)
        cp_q = pltpu.async_copy(qi.at[idxq], qi_v, sem)
        cp_e1.wait()

    def pred_step(k, carry):
        sl = pl.ds(k * LANES, LANES)
        po_v[sl] = _logsig(po_v[sl])
        return carry

    with jax.named_scope("pred"):
        lax.fori_loop(0, BPW // LANES, pred_step, 0, unroll=2)
    with jax.named_scope("gather_e2"):
        cp_e2.wait()
    with jax.named_scope("pred2"):
        lax.fori_loop(BPW // LANES, 2 * BPW // LANES, pred_step, 0, unroll=2)

    with jax.named_scope("gather_q"):
        cp_q.wait()
        cp_tab.wait()

    def compute(k, carry):
        acc_i, acc_p = carry
        t = k // 8
        u = k - t * 8
        pq = t * 256 + u * LANES

        q0 = pl.ds(pq, LANES)
        q1 = pl.ds(pq + 128, LANES)
        q2 = pl.ds(pq + 1024, LANES)
        q3 = pl.ds(pq + 1152, LANES)
        a0 = qi_v[q0]
        a1 = qi_v[q1]
        a2 = qi_v[q2]
        a3 = qi_v[q3]
        pld = pidx_v[pl.ds(k * LANES, LANES)]
        b0 = plsc.load_gather(qp_tab, [pld])
        b1 = plsc.load_gather(qp_tab, [pld + 1000])
        b2 = plsc.load_gather(qp_tab, [pld + 2000])
        b3 = plsc.load_gather(qp_tab, [pld + 3000])
        acc_i = acc_i + a0 * a0 + a1 * a1 + a2 * a2 + a3 * a3
        acc_p = acc_p + b0 * b0 + b1 * b1 + b2 * b2 + b3 * b3
        x0 = a0 + b0
        x1 = a1 + b1
        x2 = a2 + b2
        x3 = a3 + b3
        m01 = jnp.maximum(x0, x1)
        l01 = m01 + _log1p(jnp.exp(jnp.minimum(x0, x1) - m01))
        m23 = jnp.maximum(x2, x3)
        l23 = m23 + _log1p(jnp.exp(jnp.minimum(x2, x3) - m23))
        qi_v[q0] = x0 - l01
        qi_v[q1] = x1 - l01
        qi_v[q2] = x2 - l23
        qi_v[q3] = x3 - l23
        return acc_i, acc_p

    zero = jnp.zeros((LANES,), jnp.float32)
    with jax.named_scope("compute"):
        acc_i, acc_p = lax.fori_loop(0, BPW // LANES, compute, (zero, zero),
                                     unroll=2)

    pltpu.sync_copy(po_v, pred_out.at[pl.ds(wid * 2 * BPW, 2 * BPW)])
    pltpu.sync_copy(qi_v.at[pl.ds(0, 2 * BPW)], q_out.at[pl.ds(wid * 2 * BPW, 2 * BPW)])
    pltpu.sync_copy(qi_v.at[pl.ds(2 * BPW, 2 * BPW)],
                    q_out.at[pl.ds(2 * BATCH + wid * 2 * BPW, 2 * BPW)])

    acc_v[0, :] = acc_i * (BETA * 0.5)
    acc_v[1, :] = acc_p * (ALPHA * 0.5)
    pltpu.sync_copy(acc_v, shared.at[sid])
    plsc.subcore_barrier()

    @pl.when(sid == 0)
    def _():
        pltpu.sync_copy(shared, red_v)
        ti = red_v[0, 0, :]
        tp = red_v[0, 1, :]
        for s in range(1, NUM_SUBCORES):
            ti = ti + red_v[s, 0, :]
            tp = tp + red_v[s, 1, :]
        out_v[0, :] = jnp.full((LANES,), jnp.sum(ti), jnp.float32)
        out_v[1, :] = jnp.full((LANES,), jnp.sum(tp), jnp.float32)
        pltpu.sync_copy(out_v, reg_out.at[cid])


_sc_call = pl.kernel(
    _body,
    mesh=plsc.VectorSubcoreMesh(core_axis_name="c", subcore_axis_name="s"),
    compiler_params=pltpu.CompilerParams(
        needs_layout_passes=False,
        use_tc_tiling_on_sc=False,
        skip_device_barrier=True,
    ),
    out_type=[
        jax.ShapeDtypeStruct((BATCH * 2,), jnp.float32),
        jax.ShapeDtypeStruct((BATCH * 4,), jnp.float32),
        jax.ShapeDtypeStruct((NUM_CORES, 2, LANES), jnp.float32),
    ],
    scratch_types=[
        pltpu.VMEM((BPW,), jnp.int32),
        pltpu.VMEM((BPW,), jnp.int32),
        pltpu.VMEM((2 * BPW,), jnp.int32),
        pltpu.VMEM((4 * BPW,), jnp.int32),
        pltpu.VMEM((4000,), jnp.float32),
        pltpu.VMEM((2 * BPW,), jnp.float32),
        pltpu.VMEM((4 * BPW,), jnp.float32),
        pltpu.VMEM((2, LANES), jnp.float32),
        pltpu.VMEM((NUM_SUBCORES, 2, LANES), jnp.float32),
        pltpu.VMEM((2, LANES), jnp.float32),
        pltpu.VMEM_SHARED((NUM_SUBCORES, 2, LANES), jnp.float32),
        pltpu.SemaphoreType.DMA,
        pltpu.SemaphoreType.DMA,
    ],
)


def kernel(instances, predictors, labels, instance_emb, q_i_emb, q_p_emb):
    del labels
    pred, q, reg = _sc_call(
        instances, predictors,
        instance_emb.T.reshape(-1), q_i_emb.T.reshape(-1), q_p_emb.T.reshape(-1),
    )
    predictions = pred.reshape(128, 2, 128).transpose(0, 2, 1).reshape(BATCH, 2)
    q_params = q.reshape(2, 128, 2, 128).transpose(1, 3, 0, 2).reshape(BATCH, 2, 2)
    reg_i = reg[0, 0, 0] + reg[1, 0, 0]
    reg_p = reg[0, 1, 0] + reg[1, 1, 0]
    return predictions, q_params, reg_i, reg_p
